# Initial kernel scaffold; baseline (speedup 1.0000x reference)
import functools

import jax
import jax.numpy as jnp
from jax import lax
from jax.experimental import pallas as pl
from jax.experimental.pallas import tpu as pltpu
from jax.experimental.pallas import tpu_sc as plsc

_D = 77
_B = 4096 * 77
_NC, _NS = 2, 16
_NW = _NC * _NS
_BW = _B // _NW
_CHUNK = 128
_NCHUNK = _BW // _CHUNK

_mesh = plsc.VectorSubcoreMesh(core_axis_name="c", subcore_axis_name="s")


@functools.partial(
    pl.kernel,
    out_type=jax.ShapeDtypeStruct((_B, _D), jnp.float32),
    mesh=_mesh,
    scratch_types=[
        pltpu.VMEM((_NCHUNK, _CHUNK), jnp.int32),
        pltpu.VMEM((_CHUNK, _D), jnp.float32),
        pltpu.SemaphoreType.DMA,
    ],
)
def _gather(idx_hbm, table_hbm, out_hbm, idx_v, rows_v, sem):
    wid = lax.axis_index("s") * _NC + lax.axis_index("c")
    base = wid * _BW
    pltpu.sync_copy(idx_hbm.at[wid], idx_v)

    @pl.loop(0, _NCHUNK)
    def _chunk(c):
        pltpu.async_copy(table_hbm.at[idx_v.at[c]], rows_v, sem).wait()
        pltpu.sync_copy(rows_v, out_hbm.at[pl.ds(base + c * _CHUNK, _CHUNK)])


def kernel(x, table):
    idx = x.reshape(_NW, _NCHUNK, _CHUNK).astype(jnp.int32)
    out = _gather(idx, table)
    return out.reshape(x.shape[0], x.shape[1], _D)


# speedup vs baseline: 3.6512x; 3.6512x over previous
import functools

import jax
import jax.numpy as jnp
from jax import lax
from jax.experimental import pallas as pl
from jax.experimental.pallas import tpu as pltpu
from jax.experimental.pallas import tpu_sc as plsc

_D = 77
_B = 4096 * 77
_NC, _NS = 2, 16
_NW = _NC * _NS
_BW = _B // _NW
_CHUNK = 128
_NCHUNK = _BW // _CHUNK

_mesh = plsc.VectorSubcoreMesh(core_axis_name="c", subcore_axis_name="s")


@functools.partial(
    pl.kernel,
    out_type=jax.ShapeDtypeStruct((_B, 128), jnp.float32),
    mesh=_mesh,
    scratch_types=[
        pltpu.VMEM((_NCHUNK, _CHUNK), jnp.int32),
        pltpu.VMEM((_CHUNK, 128), jnp.float32),
        pltpu.SemaphoreType.DMA,
    ],
)
def _gather(idx_hbm, table_hbm, out_hbm, idx_v, rows_v, sem):
    wid = lax.axis_index("s") * _NC + lax.axis_index("c")
    base = wid * _BW
    pltpu.sync_copy(idx_hbm.at[wid], idx_v)

    @pl.loop(0, _NCHUNK)
    def _chunk(c):
        pltpu.async_copy(table_hbm.at[idx_v.at[c]], rows_v, sem).wait()
        pltpu.sync_copy(rows_v, out_hbm.at[pl.ds(base + c * _CHUNK, _CHUNK)])


def kernel(x, table):
    idx = x.reshape(_NW, _NCHUNK, _CHUNK).astype(jnp.int32)
    table_p = jnp.pad(table, ((0, 0), (0, 128 - _D)))
    out = _gather(idx, table_p)
    return out[:, :_D].reshape(x.shape[0], x.shape[1], _D)


# speedup vs baseline: 3.8067x; 1.0426x over previous
import functools

import jax
import jax.numpy as jnp
from jax import lax
from jax.experimental import pallas as pl
from jax.experimental.pallas import tpu as pltpu
from jax.experimental.pallas import tpu_sc as plsc

_D = 77
_B = 4096 * 77
_NC, _NS = 2, 16
_NW = _NC * _NS
_BW = _B // _NW
_CHUNK = 112
_NCHUNK = _BW // _CHUNK
_NBUF = 8

_mesh = plsc.VectorSubcoreMesh(core_axis_name="c", subcore_axis_name="s")


@functools.partial(
    pl.kernel,
    out_type=jax.ShapeDtypeStruct((_B, 128), jnp.float32),
    mesh=_mesh,
    scratch_types=[
        pltpu.VMEM((_NCHUNK, _CHUNK), jnp.int32),
        pltpu.VMEM((_NBUF, _CHUNK, 128), jnp.float32),
    ]
    + [pltpu.SemaphoreType.DMA] * (2 * _NBUF),
)
def _gather(idx_hbm, table_hbm, out_hbm, idx_v, bufs, *sems):
    gsem, wsem = sems[:_NBUF], sems[_NBUF:]
    wid = lax.axis_index("s") * _NC + lax.axis_index("c")
    base = wid * _BW
    pltpu.sync_copy(idx_hbm.at[wid], idx_v)

    for b in range(_NBUF):
        pltpu.async_copy(table_hbm.at[idx_v.at[b]], bufs.at[b], gsem[b])

    @pl.loop(0, _NCHUNK, step=_NBUF)
    def _group(g):
        for b in range(_NBUF):
            c = g + b
            pltpu.make_async_copy(
                table_hbm.at[idx_v.at[b]], bufs.at[b], gsem[b]
            ).wait()
            pltpu.async_copy(
                bufs.at[b], out_hbm.at[pl.ds(base + c * _CHUNK, _CHUNK)], wsem[b]
            )
        for b in range(_NBUF):
            c = g + b
            pltpu.make_async_copy(
                bufs.at[b], out_hbm.at[pl.ds(base + c * _CHUNK, _CHUNK)], wsem[b]
            ).wait()

            @pl.when(c + _NBUF < _NCHUNK)
            def _():
                pltpu.async_copy(
                    table_hbm.at[idx_v.at[c + _NBUF]], bufs.at[b], gsem[b]
                )


def kernel(x, table):
    idx = x.reshape(_NW, _NCHUNK, _CHUNK).astype(jnp.int32)
    table_p = jnp.pad(table, ((0, 0), (0, 128 - _D)))
    out = _gather(idx, table_p)
    return out[:, :_D].reshape(x.shape[0], x.shape[1], _D)


# speedup vs baseline: 4.4048x; 1.1571x over previous
import functools

import jax
import jax.numpy as jnp
from jax import lax
from jax.experimental import pallas as pl
from jax.experimental.pallas import tpu as pltpu
from jax.experimental.pallas import tpu_sc as plsc

_D = 77
_N = 4096
_V = 1000
_TF = _V * _D
_TFP = 77056
_NC, _NS = 2, 16
_NW = _NC * _NS
_NBLK = _N // _NW
_NBUF = 2
_OFFS = (0, 16, 32, 48, 61)

_mesh = plsc.VectorSubcoreMesh(core_axis_name="c", subcore_axis_name="s")


@functools.partial(
    pl.kernel,
    out_type=jax.ShapeDtypeStruct((_N, _D, _D), jnp.float32),
    mesh=_mesh,
    scratch_types=[
        pltpu.VMEM((_TFP,), jnp.float32),
        pltpu.VMEM((_NBLK, _D), jnp.int32),
        pltpu.VMEM((_NBUF, _D, _D), jnp.float32),
    ]
    + [pltpu.SemaphoreType.DMA] * _NBUF,
)
def _embed(x_hbm, table_hbm, out_hbm, table_v, idx_v, bufs, *wsem):
    wid = lax.axis_index("s") * _NC + lax.axis_index("c")
    ibase = wid * _NBLK
    pltpu.sync_copy(table_hbm, table_v)
    pltpu.sync_copy(x_hbm.at[pl.ds(ibase, _NBLK)], idx_v)

    def build(k, b):
        ivs = [idx_v[k, pl.ds(o, 16)] for o in _OFFS]
        for j in range(_D):
            wv = min(j // 16, 4)
            t = ivs[wv][j - _OFFS[wv]] * _D
            for o in _OFFS:
                bufs[b, j, pl.ds(o, 16)] = table_v[pl.ds(t + o, 16)]

    for b in range(_NBUF):
        build(b, b)
        pltpu.async_copy(bufs.at[b], out_hbm.at[ibase + b], wsem[b])

    @pl.loop(0, _NBLK - _NBUF, step=_NBUF)
    def _group(k0):
        for b in range(_NBUF):
            kn = k0 + b + _NBUF
            pltpu.make_async_copy(bufs.at[b], out_hbm.at[ibase], wsem[b]).wait()
            build(kn, b)
            pltpu.async_copy(bufs.at[b], out_hbm.at[ibase + kn], wsem[b])

    for b in range(_NBUF):
        pltpu.make_async_copy(bufs.at[b], out_hbm.at[ibase], wsem[b]).wait()


def kernel(x, table):
    table_flat = jnp.pad(table.reshape(-1), (0, _TFP - _TF))
    return _embed(x.astype(jnp.int32), table_flat)
